# Initial kernel scaffold; baseline (speedup 1.0000x reference)
import jax
import jax.numpy as jnp
from jax.experimental import pallas as pl


def kernel(card_ids, costs, rarities, types, upgrades, card_id_table, cost_table, rarity_table, type_table, upgrade_table, W, b):
    raise NotImplementedError("write your pallas kernel here")


# speedup vs baseline: 13.4980x; 13.4980x over previous
import functools

import jax
import jax.numpy as jnp
from jax import lax
from jax.experimental import pallas as pl
from jax.experimental.pallas import tpu as pltpu
from jax.experimental.pallas import tpu_sc as plsc

EMB = 128
N_COST, N_RAR, N_TYPE, N_UPG = 11, 5, 4, 21
N_COMB = N_COST * N_RAR * N_TYPE * N_UPG


def _prep_body(cardt_ref, costt_ref, rart_ref, typet_ref, upgt_ref, w_ref, b_ref,
               costs_ref, rars_ref, typs_ref, upgs_ref,
               pcard_ref, comb_ref, cidx_ref):
    w = w_ref[...]
    pcard_ref[...] = jnp.dot(cardt_ref[...], w[0:64, :],
                             preferred_element_type=jnp.float32)
    pc = jnp.dot(costt_ref[...], w[64:72, :], preferred_element_type=jnp.float32)
    pr = jnp.dot(rart_ref[...], w[72:80, :], preferred_element_type=jnp.float32)
    pt = jnp.dot(typet_ref[...], w[80:88, :], preferred_element_type=jnp.float32)
    pu = jnp.dot(upgt_ref[...], w[88:96, :], preferred_element_type=jnp.float32)
    tu = (pt[:, None, :] + pu[None, :, :]).reshape(N_TYPE * N_UPG, EMB)
    rtu = (pr[:, None, :] + tu[None, :, :]).reshape(N_RAR * N_TYPE * N_UPG, EMB)
    crtu = (pc[:, None, :] + rtu[None, :, :]).reshape(N_COMB, EMB)
    comb_ref[...] = crtu + b_ref[...]
    cidx_ref[...] = ((costs_ref[...] * N_RAR + rars_ref[...]) * N_TYPE
                     + typs_ref[...]) * N_UPG + upgs_ref[...]


def _sc_gather_add(n_rows, chunk, n_chunks_per_worker, nc, ns):
    mesh = plsc.VectorSubcoreMesh(core_axis_name="c", subcore_axis_name="s")
    rows_per_worker = chunk * n_chunks_per_worker

    @functools.partial(
        pl.kernel,
        mesh=mesh,
        out_type=jax.ShapeDtypeStruct((n_rows, EMB), jnp.float32),
        scratch_types=[
            pltpu.VMEM((chunk,), jnp.int32),
            pltpu.VMEM((chunk,), jnp.int32),
            pltpu.VMEM((chunk, EMB), jnp.float32),
            pltpu.VMEM((chunk, EMB), jnp.float32),
            pltpu.SemaphoreType.DMA,
            pltpu.SemaphoreType.DMA,
        ],
    )
    def body(pcard_hbm, comb_hbm, cids_hbm, cidx_hbm, out_hbm,
             ia, ib, ra, rb, s1, s2):
        wid = lax.axis_index("s") * nc + lax.axis_index("c")
        w_base = wid * rows_per_worker

        def run_chunk(j, carry):
            base = w_base + j * chunk
            pltpu.sync_copy(cids_hbm.at[pl.ds(base, chunk)], ia)
            pltpu.sync_copy(cidx_hbm.at[pl.ds(base, chunk)], ib)
            ca = pltpu.async_copy(pcard_hbm.at[ia], ra, s1)
            cb = pltpu.async_copy(comb_hbm.at[ib], rb, s2)
            ca.wait()
            cb.wait()

            def add_row(r, c2):
                for c in range(EMB // 16):
                    sl = pl.ds(c * 16, 16)
                    ra[r, sl] = ra[r, sl] + rb[r, sl]
                return c2

            lax.fori_loop(0, chunk, add_row, 0)
            pltpu.sync_copy(ra, out_hbm.at[pl.ds(base, chunk)])
            return carry

        lax.fori_loop(0, n_chunks_per_worker, run_chunk, 0)

    return body


def kernel(card_ids, costs, rarities, types, upgrades,
           card_id_table, cost_table, rarity_table, type_table, upgrade_table,
           W, b):
    B, L = card_ids.shape
    n_rows = B * L

    pcard, comb, cidx = pl.pallas_call(
        _prep_body,
        out_shape=[
            jax.ShapeDtypeStruct((card_id_table.shape[0], EMB), jnp.float32),
            jax.ShapeDtypeStruct((N_COMB, EMB), jnp.float32),
            jax.ShapeDtypeStruct((B, L), jnp.int32),
        ],
    )(card_id_table, cost_table, rarity_table, type_table, upgrade_table,
      W, b.reshape(1, EMB),
      costs.astype(jnp.int32), rarities.astype(jnp.int32),
      types.astype(jnp.int32), upgrades.astype(jnp.int32))

    info = plsc.get_sparse_core_info()
    nc, ns = info.num_cores, info.num_subcores
    nw = nc * ns
    chunk = 128
    n_chunks_per_worker = n_rows // (nw * chunk)
    assert n_chunks_per_worker * nw * chunk == n_rows

    sc = _sc_gather_add(n_rows, chunk, n_chunks_per_worker, nc, ns)
    out = sc(pcard, comb,
             card_ids.reshape(n_rows).astype(jnp.int32),
             cidx.reshape(n_rows))
    return out.reshape(B, L, EMB)


# speedup vs baseline: 15.1326x; 1.1211x over previous
import functools

import jax
import jax.numpy as jnp
from jax import lax
from jax.experimental import pallas as pl
from jax.experimental.pallas import tpu as pltpu
from jax.experimental.pallas import tpu_sc as plsc

EMB = 128
N_COST, N_RAR, N_TYPE, N_UPG = 11, 5, 4, 21
N_COMB = N_COST * N_RAR * N_TYPE * N_UPG


def _prep_body(cardt_ref, costt_ref, rart_ref, typet_ref, upgt_ref, w_ref, b_ref,
               costs_ref, rars_ref, typs_ref, upgs_ref,
               pcard_ref, comb_ref, cidx_ref):
    w = w_ref[...]
    pcard_ref[...] = jnp.dot(cardt_ref[...], w[0:64, :],
                             preferred_element_type=jnp.float32)
    pc = jnp.dot(costt_ref[...], w[64:72, :], preferred_element_type=jnp.float32)
    pr = jnp.dot(rart_ref[...], w[72:80, :], preferred_element_type=jnp.float32)
    pt = jnp.dot(typet_ref[...], w[80:88, :], preferred_element_type=jnp.float32)
    pu = jnp.dot(upgt_ref[...], w[88:96, :], preferred_element_type=jnp.float32)
    tu = (pt[:, None, :] + pu[None, :, :]).reshape(N_TYPE * N_UPG, EMB)
    rtu = (pr[:, None, :] + tu[None, :, :]).reshape(N_RAR * N_TYPE * N_UPG, EMB)
    crtu = (pc[:, None, :] + rtu[None, :, :]).reshape(N_COMB, EMB)
    comb_ref[...] = crtu + b_ref[...]
    cidx_ref[...] = ((costs_ref[...] * N_RAR + rars_ref[...]) * N_TYPE
                     + typs_ref[...]) * N_UPG + upgs_ref[...]


def _sc_gather_add(n_rows, chunk, n_chunks_per_worker, nc, ns):
    mesh = plsc.VectorSubcoreMesh(core_axis_name="c", subcore_axis_name="s")
    rows_per_worker = chunk * n_chunks_per_worker
    ncw = n_chunks_per_worker
    G = 4
    n_groups = ncw // G
    assert n_groups * G == ncw and n_groups % 2 == 0

    @functools.partial(
        pl.kernel,
        mesh=mesh,
        out_type=jax.ShapeDtypeStruct((n_rows, EMB), jnp.float32),
        scratch_types=[
            [pltpu.VMEM((G, chunk), jnp.int32)] * 2,
            [pltpu.VMEM((G, chunk), jnp.int32)] * 2,
            [pltpu.VMEM((chunk, EMB), jnp.float32)] * 2,
            [pltpu.VMEM((chunk, EMB), jnp.float32)] * 2,
            [pltpu.VMEM((chunk, EMB), jnp.float32)] * 2,
            [pltpu.SemaphoreType.DMA] * 2,
            [pltpu.SemaphoreType.DMA] * 2,
            [pltpu.SemaphoreType.DMA] * 2,
            [pltpu.SemaphoreType.DMA] * 2,
            [pltpu.SemaphoreType.DMA] * 2,
        ],
    )
    def body(pcard_hbm, comb_hbm, cids_hbm, cidx_hbm, out_hbm,
             ia, ib, ra, rb, ob, sga, sgb, sw, sia, sib):
        wid = lax.axis_index("s") * nc + lax.axis_index("c")
        w_chunk = wid * ncw
        w_base = wid * rows_per_worker

        def idx_issue(g, s):
            sl = pl.ds(w_chunk + g * G, G)
            pltpu.async_copy(cids_hbm.at[sl], ia[s], sia[s])
            pltpu.async_copy(cidx_hbm.at[sl], ib[s], sib[s])

        def idx_drain(g, s):
            sl = pl.ds(w_chunk + g * G, G)
            pltpu.make_async_copy(cids_hbm.at[sl], ia[s], sia[s]).wait()
            pltpu.make_async_copy(cidx_hbm.at[sl], ib[s], sib[s]).wait()

        def gather_issue(s, row, t):
            pltpu.async_copy(pcard_hbm.at[ia[s].at[row]], ra[t], sga[t])
            pltpu.async_copy(comb_hbm.at[ib[s].at[row]], rb[t], sgb[t])

        def gather_drain(s, row, t):
            pltpu.make_async_copy(pcard_hbm.at[ia[s].at[row]], ra[t], sga[t]).wait()
            pltpu.make_async_copy(comb_hbm.at[ib[s].at[row]], rb[t], sgb[t]).wait()

        def add_into(t):
            def add_row(r, carry):
                for c in range(EMB // 16):
                    sl = pl.ds(c * 16, 16)
                    ob[t][r, sl] = ra[t][r, sl] + rb[t][r, sl]
                return carry
            lax.fori_loop(0, chunk, add_row, 0)

        def write_issue(c, t):
            pltpu.async_copy(ob[t], out_hbm.at[pl.ds(w_base + c * chunk, chunk)],
                             sw[t])

        def write_drain(c, t):
            pltpu.make_async_copy(
                ob[t], out_hbm.at[pl.ds(w_base + c * chunk, chunk)], sw[t]
            ).wait()

        idx_issue(0, 0)
        idx_drain(0, 0)
        idx_issue(1, 1)
        gather_issue(0, 0, 0)
        gather_issue(0, 1, 1)

        def run_steps(m, first):
            for p in range(2):
                for k in range(G):
                    c = 8 * m + 4 * p + k
                    t = k % 2
                    s2 = (p + (k + 2) // 4) % 2
                    row2 = (4 * p + k + 2) % 4
                    gather_drain(p, k, t)
                    if first:
                        if 4 * p + k >= 2:
                            write_drain(c - 2, t)
                        add_into(t)
                        if k == 2:
                            idx_drain(2 * m + p + 1, s2)
                        gather_issue(s2, row2, t)
                    else:
                        write_drain(c - 2, t)
                        add_into(t)
                        if k == 2:
                            @pl.when(c + 2 < ncw)
                            def _():
                                idx_drain(2 * m + p + 1, s2)
                                gather_issue(s2, row2, t)
                        else:
                            @pl.when(c + 2 < ncw)
                            def _():
                                gather_issue(s2, row2, t)
                    write_issue(c, t)
                if first:
                    idx_issue(2 * m + p + 2, p)
                else:
                    @pl.when(2 * m + p + 2 < n_groups)
                    def _():
                        idx_issue(2 * m + p + 2, p)

        run_steps(0, True)

        def loop_body(m, carry):
            run_steps(m, False)
            return carry

        lax.fori_loop(1, n_groups // 2, loop_body, 0)
        write_drain(ncw - 2, 0)
        write_drain(ncw - 1, 1)

    return body


def kernel(card_ids, costs, rarities, types, upgrades,
           card_id_table, cost_table, rarity_table, type_table, upgrade_table,
           W, b):
    B, L = card_ids.shape
    n_rows = B * L

    pcard, comb, cidx = pl.pallas_call(
        _prep_body,
        out_shape=[
            jax.ShapeDtypeStruct((card_id_table.shape[0], EMB), jnp.float32),
            jax.ShapeDtypeStruct((N_COMB, EMB), jnp.float32),
            jax.ShapeDtypeStruct((B, L), jnp.int32),
        ],
    )(card_id_table, cost_table, rarity_table, type_table, upgrade_table,
      W, b.reshape(1, EMB),
      costs.astype(jnp.int32), rarities.astype(jnp.int32),
      types.astype(jnp.int32), upgrades.astype(jnp.int32))

    info = plsc.get_sparse_core_info()
    nc, ns = info.num_cores, info.num_subcores
    nw = nc * ns
    chunk = 128
    n_chunks_per_worker = n_rows // (nw * chunk)
    assert n_chunks_per_worker * nw * chunk == n_rows

    sc = _sc_gather_add(n_rows, chunk, n_chunks_per_worker, nc, ns)
    out = sc(pcard, comb,
             card_ids.reshape(n_rows // chunk, chunk).astype(jnp.int32),
             cidx.reshape(n_rows // chunk, chunk))
    return out.reshape(B, L, EMB)


# speedup vs baseline: 16.5087x; 1.0909x over previous
import functools

import jax
import jax.numpy as jnp
from jax import lax
from jax.experimental import pallas as pl
from jax.experimental.pallas import tpu as pltpu
from jax.experimental.pallas import tpu_sc as plsc

EMB = 128
N_COST, N_RAR, N_TYPE, N_UPG = 11, 5, 4, 21
N_COMB = N_COST * N_RAR * N_TYPE * N_UPG


N_OH = 560


def _prep_body(cardt_ref, costt_ref, rart_ref, typet_ref, upgt_ref, w_ref, b_ref,
               costs_ref, rars_ref, typs_ref, upgs_ref,
               pcard_ref, comb_ref, cidx_ref, pall_ref):
    w = w_ref[...]
    pcard = jnp.dot(cardt_ref[...], w[0:64, :],
                    preferred_element_type=jnp.float32)
    pcard_ref[...] = pcard
    pc = jnp.dot(costt_ref[...], w[64:72, :], preferred_element_type=jnp.float32)
    pr = jnp.dot(rart_ref[...], w[72:80, :], preferred_element_type=jnp.float32)
    pt = jnp.dot(typet_ref[...], w[80:88, :], preferred_element_type=jnp.float32)
    pu = jnp.dot(upgt_ref[...], w[88:96, :], preferred_element_type=jnp.float32)
    tu = (pt[:, None, :] + pu[None, :, :]).reshape(N_TYPE * N_UPG, EMB)
    rtu = (pr[:, None, :] + tu[None, :, :]).reshape(N_RAR * N_TYPE * N_UPG, EMB)
    crtu = (pc[:, None, :] + rtu[None, :, :]).reshape(N_COMB, EMB)
    comb_ref[...] = crtu + b_ref[...]
    cidx_ref[...] = ((costs_ref[...] * N_RAR + rars_ref[...]) * N_TYPE
                     + typs_ref[...]) * N_UPG + upgs_ref[...]
    pall_ref[...] = jnp.concatenate(
        [pcard, pc, pr, pt, pu, jnp.zeros((N_OH - 554, EMB), jnp.float32)],
        axis=0).astype(jnp.bfloat16)


def _tc_onehot_body(cids_ref, costs_ref, rars_ref, typs_ref, upgs_ref,
                    pall_ref, b_ref, out_ref):
    n = out_ref.shape[0]
    iota = lax.broadcasted_iota(jnp.int32, (n, N_OH), 1)
    m = iota == cids_ref[0, 0, :][:, None]
    m = m | (iota == (costs_ref[0, 0, :] + 513)[:, None])
    m = m | (iota == (rars_ref[0, 0, :] + 524)[:, None])
    m = m | (iota == (typs_ref[0, 0, :] + 529)[:, None])
    m = m | (iota == (upgs_ref[0, 0, :] + 533)[:, None])
    oh = m.astype(jnp.bfloat16)
    out_ref[...] = (jnp.dot(oh, pall_ref[...],
                            preferred_element_type=jnp.float32) + b_ref[...])


def _sc_gather_add(n_rows, chunk, n_chunks_per_worker, nc, ns):
    mesh = plsc.VectorSubcoreMesh(core_axis_name="c", subcore_axis_name="s")
    rows_per_worker = chunk * n_chunks_per_worker
    ncw = n_chunks_per_worker
    G = 4
    n_groups = ncw // G
    assert n_groups * G == ncw and n_groups % 2 == 0

    @functools.partial(
        pl.kernel,
        mesh=mesh,
        out_type=jax.ShapeDtypeStruct((n_rows, EMB), jnp.float32),
        scratch_types=[
            [pltpu.VMEM((G, chunk), jnp.int32)] * 2,
            [pltpu.VMEM((G, chunk), jnp.int32)] * 2,
            [pltpu.VMEM((chunk, EMB), jnp.float32)] * 4,
            [pltpu.VMEM((chunk, EMB), jnp.float32)] * 2,
            [pltpu.SemaphoreType.DMA] * 2,
            [pltpu.SemaphoreType.DMA] * 2,
            [pltpu.SemaphoreType.DMA] * 4,
            [pltpu.SemaphoreType.DMA] * 2,
            [pltpu.SemaphoreType.DMA] * 4,
            pltpu.VMEM_SHARED((513, EMB), jnp.float32),
            pltpu.VMEM_SHARED((N_COMB, EMB), jnp.float32),
        ],
    )
    def body(pcard_hbm, comb_hbm, cids_hbm, cidx_hbm, out_hbm,
             ia, ib, ga, gb, sia, sib, sga, sgb, sw, sh_pcard, sh_comb):
        wid = lax.axis_index("s") * nc + lax.axis_index("c")
        w_chunk = wid * ncw
        w_base = wid * rows_per_worker

        @pl.when(lax.axis_index("s") == 0)
        def _():
            pltpu.sync_copy(pcard_hbm, sh_pcard)
            pltpu.sync_copy(comb_hbm, sh_comb)

        plsc.subcore_barrier()

        def idx_issue(g, s):
            sl = pl.ds(w_chunk + g * G, G)
            pltpu.async_copy(cids_hbm.at[sl], ia[s], sia[s])
            pltpu.async_copy(cidx_hbm.at[sl], ib[s], sib[s])

        def idx_drain(g, s):
            sl = pl.ds(w_chunk + g * G, G)
            pltpu.make_async_copy(cids_hbm.at[sl], ia[s], sia[s]).wait()
            pltpu.make_async_copy(cidx_hbm.at[sl], ib[s], sib[s]).wait()

        def gather_issue(s, row, t4, t2):
            pltpu.async_copy(sh_pcard.at[ia[s].at[row]], ga[t4], sga[t4])
            pltpu.async_copy(sh_comb.at[ib[s].at[row]], gb[t2], sgb[t2])

        def gather_drain(s, row, t4, t2):
            pltpu.make_async_copy(sh_pcard.at[ia[s].at[row]], ga[t4],
                                  sga[t4]).wait()
            pltpu.make_async_copy(sh_comb.at[ib[s].at[row]], gb[t2],
                                  sgb[t2]).wait()

        def add_into(t4, t2):
            @plsc.parallel_loop(0, chunk, 1, unroll=4)
            def _(r):
                for c in range(EMB // 16):
                    sl = pl.ds(c * 16, 16)
                    plsc.addupdate(ga[t4].at[r, sl], gb[t2][r, sl])

        def write_issue(c, t4):
            pltpu.async_copy(ga[t4],
                             out_hbm.at[pl.ds(w_base + c * chunk, chunk)],
                             sw[t4])

        def write_drain(c, t4):
            pltpu.make_async_copy(
                ga[t4], out_hbm.at[pl.ds(w_base + c * chunk, chunk)], sw[t4]
            ).wait()

        idx_issue(0, 0)
        idx_drain(0, 0)
        idx_issue(1, 1)
        gather_issue(0, 0, 0, 0)
        gather_issue(0, 1, 1, 1)

        def run_steps(m, first):
            for p in range(2):
                for k in range(G):
                    c = 8 * m + 4 * p + k
                    t4 = k
                    t2 = k % 2
                    t4n = (k + 2) % 4
                    s2 = (p + (k + 2) // 4) % 2
                    row2 = (4 * p + k + 2) % 4
                    gather_drain(p, k, t4, t2)
                    add_into(t4, t2)
                    if first:
                        if 4 * p + k >= 2:
                            write_drain(c - 2, t4n)
                        if k == 2:
                            idx_drain(2 * m + p + 1, s2)
                        gather_issue(s2, row2, t4n, t2)
                    else:
                        write_drain(c - 2, t4n)
                        if k == 2:
                            @pl.when(c + 2 < ncw)
                            def _():
                                idx_drain(2 * m + p + 1, s2)
                                gather_issue(s2, row2, t4n, t2)
                        else:
                            @pl.when(c + 2 < ncw)
                            def _():
                                gather_issue(s2, row2, t4n, t2)
                    write_issue(c, t4)
                if first:
                    idx_issue(2 * m + p + 2, p)
                else:
                    @pl.when(2 * m + p + 2 < n_groups)
                    def _():
                        idx_issue(2 * m + p + 2, p)

        run_steps(0, True)

        def loop_body(m, carry):
            run_steps(m, False)
            return carry

        lax.fori_loop(1, n_groups // 2, loop_body, 0)
        write_drain(ncw - 2, 2)
        write_drain(ncw - 1, 3)

    return body


def kernel(card_ids, costs, rarities, types, upgrades,
           card_id_table, cost_table, rarity_table, type_table, upgrade_table,
           W, b):
    B, L = card_ids.shape
    n_rows = B * L

    pcard, comb, cidx, pall = pl.pallas_call(
        _prep_body,
        out_shape=[
            jax.ShapeDtypeStruct((card_id_table.shape[0], EMB), jnp.float32),
            jax.ShapeDtypeStruct((N_COMB, EMB), jnp.float32),
            jax.ShapeDtypeStruct((B, L), jnp.int32),
            jax.ShapeDtypeStruct((N_OH, EMB), jnp.bfloat16),
        ],
    )(card_id_table, cost_table, rarity_table, type_table, upgrade_table,
      W, b.reshape(1, EMB),
      costs.astype(jnp.int32), rarities.astype(jnp.int32),
      types.astype(jnp.int32), upgrades.astype(jnp.int32))

    info = plsc.get_sparse_core_info()
    nc, ns = info.num_cores, info.num_subcores
    nw = nc * ns
    chunk = 80
    ncw = 216
    n_sc = nw * chunk * ncw
    n_tc = n_rows - n_sc
    tc_blk = 4096
    assert n_tc % tc_blk == 0

    cids_flat = card_ids.reshape(n_rows).astype(jnp.int32)
    cidx_flat = cidx.reshape(n_rows)

    sc = _sc_gather_add(n_sc, chunk, ncw, nc, ns)
    out_sc = sc(pcard, comb,
                cids_flat[:n_sc].reshape(n_sc // chunk, chunk),
                cidx_flat[:n_sc].reshape(n_sc // chunk, chunk))

    def tc_idx(a):
        return a.reshape(n_rows)[n_sc:].reshape(n_tc // tc_blk, 1, tc_blk)

    grid = (n_tc // tc_blk,)
    idx_spec = pl.BlockSpec((1, 1, tc_blk), lambda i: (i, 0, 0))
    out_tc = pl.pallas_call(
        _tc_onehot_body,
        grid=grid,
        in_specs=[idx_spec, idx_spec, idx_spec, idx_spec, idx_spec,
                  pl.BlockSpec((N_OH, EMB), lambda i: (0, 0)),
                  pl.BlockSpec((1, EMB), lambda i: (0, 0))],
        out_specs=pl.BlockSpec((tc_blk, EMB), lambda i: (i, 0)),
        out_shape=jax.ShapeDtypeStruct((n_tc, EMB), jnp.float32),
    )(tc_idx(cids_flat), tc_idx(costs.astype(jnp.int32)),
      tc_idx(rarities.astype(jnp.int32)), tc_idx(types.astype(jnp.int32)),
      tc_idx(upgrades.astype(jnp.int32)), pall, b.reshape(1, EMB))

    out = jnp.concatenate([out_sc, out_tc], axis=0)
    return out.reshape(B, L, EMB)


# speedup vs baseline: 33.2279x; 2.0128x over previous
import functools

import jax
import jax.numpy as jnp
from jax import lax
from jax.experimental import pallas as pl
from jax.experimental.pallas import tpu as pltpu
from jax.experimental.pallas import tpu_sc as plsc

EMB = 128
N_COST, N_RAR, N_TYPE, N_UPG = 11, 5, 4, 21
N_COMB = N_COST * N_RAR * N_TYPE * N_UPG


def _prep_body(cardt_ref, costt_ref, rart_ref, typet_ref, upgt_ref, w_ref, b_ref,
               costs_ref, rars_ref, typs_ref, upgs_ref,
               pcard_ref, comb_ref, cidx_ref):
    w = w_ref[...]
    pcard_ref[...] = jnp.dot(cardt_ref[...], w[0:64, :],
                             preferred_element_type=jnp.float32)
    pc = jnp.dot(costt_ref[...], w[64:72, :], preferred_element_type=jnp.float32)
    pr = jnp.dot(rart_ref[...], w[72:80, :], preferred_element_type=jnp.float32)
    pt = jnp.dot(typet_ref[...], w[80:88, :], preferred_element_type=jnp.float32)
    pu = jnp.dot(upgt_ref[...], w[88:96, :], preferred_element_type=jnp.float32)
    tu = (pt[:, None, :] + pu[None, :, :]).reshape(N_TYPE * N_UPG, EMB)
    rtu = (pr[:, None, :] + tu[None, :, :]).reshape(N_RAR * N_TYPE * N_UPG, EMB)
    crtu = (pc[:, None, :] + rtu[None, :, :]).reshape(N_COMB, EMB)
    comb_ref[...] = crtu + b_ref[...]
    cidx_ref[...] = ((costs_ref[...] * N_RAR + rars_ref[...]) * N_TYPE
                     + typs_ref[...]) * N_UPG + upgs_ref[...]


def _sc_gather_add(n_rows, chunk, n_chunks_per_worker, nc, ns):
    mesh = plsc.VectorSubcoreMesh(core_axis_name="c", subcore_axis_name="s")
    rows_per_worker = chunk * n_chunks_per_worker
    ncw = n_chunks_per_worker
    G = 4
    n_groups = ncw // G
    assert n_groups * G == ncw and n_groups % 2 == 0

    @functools.partial(
        pl.kernel,
        mesh=mesh,
        out_type=jax.ShapeDtypeStruct((n_rows, EMB), jnp.float32),
        scratch_types=[
            [pltpu.VMEM((G, chunk), jnp.int32)] * 2,
            [pltpu.VMEM((G, chunk), jnp.int32)] * 2,
            [pltpu.VMEM((chunk, EMB), jnp.float32)] * 4,
            [pltpu.VMEM((chunk, EMB), jnp.float32)] * 2,
            [pltpu.SemaphoreType.DMA] * 2,
            [pltpu.SemaphoreType.DMA] * 2,
            [pltpu.SemaphoreType.DMA] * 4,
            [pltpu.SemaphoreType.DMA] * 2,
            [pltpu.SemaphoreType.DMA] * 4,
            pltpu.VMEM_SHARED((513, EMB), jnp.float32),
            pltpu.VMEM_SHARED((N_COMB, EMB), jnp.float32),
        ],
    )
    def body(pcard_hbm, comb_hbm, cids_hbm, cidx_hbm, out_hbm,
             ia, ib, ga, gb, sia, sib, sga, sgb, sw, sh_pcard, sh_comb):
        wid = lax.axis_index("s") * nc + lax.axis_index("c")
        w_chunk = wid * ncw
        w_base = wid * rows_per_worker

        @pl.when(lax.axis_index("s") == 0)
        def _():
            pltpu.sync_copy(pcard_hbm, sh_pcard)
            pltpu.sync_copy(comb_hbm, sh_comb)

        plsc.subcore_barrier()

        def idx_issue(g, s):
            sl = pl.ds(w_chunk + g * G, G)
            pltpu.async_copy(cids_hbm.at[sl], ia[s], sia[s])
            pltpu.async_copy(cidx_hbm.at[sl], ib[s], sib[s])

        def idx_drain(g, s):
            sl = pl.ds(w_chunk + g * G, G)
            pltpu.make_async_copy(cids_hbm.at[sl], ia[s], sia[s]).wait()
            pltpu.make_async_copy(cidx_hbm.at[sl], ib[s], sib[s]).wait()

        def gather_issue(s, row, t4, t2):
            pltpu.async_copy(sh_pcard.at[ia[s].at[row]], ga[t4], sga[t4])
            pltpu.async_copy(sh_comb.at[ib[s].at[row]], gb[t2], sgb[t2])

        def gather_drain(s, row, t4, t2):
            pltpu.make_async_copy(sh_pcard.at[ia[s].at[row]], ga[t4],
                                  sga[t4]).wait()
            pltpu.make_async_copy(sh_comb.at[ib[s].at[row]], gb[t2],
                                  sgb[t2]).wait()

        def add_into(t4, t2):
            @plsc.parallel_loop(0, chunk, 1, unroll=4)
            def _(r):
                for c in range(EMB // 16):
                    sl = pl.ds(c * 16, 16)
                    plsc.addupdate(ga[t4].at[r, sl], gb[t2][r, sl])

        def write_issue(c, t4):
            pltpu.async_copy(ga[t4],
                             out_hbm.at[pl.ds(w_base + c * chunk, chunk)],
                             sw[t4])

        def write_drain(c, t4):
            pltpu.make_async_copy(
                ga[t4], out_hbm.at[pl.ds(w_base + c * chunk, chunk)], sw[t4]
            ).wait()

        idx_issue(0, 0)
        idx_drain(0, 0)
        idx_issue(1, 1)
        gather_issue(0, 0, 0, 0)
        gather_issue(0, 1, 1, 1)

        def run_steps(m, first):
            for p in range(2):
                for k in range(G):
                    c = 8 * m + 4 * p + k
                    t4 = k
                    t2 = k % 2
                    t4n = (k + 2) % 4
                    s2 = (p + (k + 2) // 4) % 2
                    row2 = (4 * p + k + 2) % 4
                    gather_drain(p, k, t4, t2)
                    add_into(t4, t2)
                    if first:
                        if 4 * p + k >= 2:
                            write_drain(c - 2, t4n)
                        if k == 2:
                            idx_drain(2 * m + p + 1, s2)
                        gather_issue(s2, row2, t4n, t2)
                    else:
                        write_drain(c - 2, t4n)
                        if k == 2:
                            @pl.when(c + 2 < ncw)
                            def _():
                                idx_drain(2 * m + p + 1, s2)
                                gather_issue(s2, row2, t4n, t2)
                        else:
                            @pl.when(c + 2 < ncw)
                            def _():
                                gather_issue(s2, row2, t4n, t2)
                    write_issue(c, t4)
                if first:
                    idx_issue(2 * m + p + 2, p)
                else:
                    @pl.when(2 * m + p + 2 < n_groups)
                    def _():
                        idx_issue(2 * m + p + 2, p)

        run_steps(0, True)

        def loop_body(m, carry):
            run_steps(m, False)
            return carry

        lax.fori_loop(1, n_groups // 2, loop_body, 0)
        write_drain(ncw - 2, 2)
        write_drain(ncw - 1, 3)

    return body


def kernel(card_ids, costs, rarities, types, upgrades,
           card_id_table, cost_table, rarity_table, type_table, upgrade_table,
           W, b):
    B, L = card_ids.shape
    n_rows = B * L

    pcard, comb, cidx = pl.pallas_call(
        _prep_body,
        out_shape=[
            jax.ShapeDtypeStruct((card_id_table.shape[0], EMB), jnp.float32),
            jax.ShapeDtypeStruct((N_COMB, EMB), jnp.float32),
            jax.ShapeDtypeStruct((B, L), jnp.int32),
        ],
    )(card_id_table, cost_table, rarity_table, type_table, upgrade_table,
      W, b.reshape(1, EMB),
      costs.astype(jnp.int32), rarities.astype(jnp.int32),
      types.astype(jnp.int32), upgrades.astype(jnp.int32))

    info = plsc.get_sparse_core_info()
    nc, ns = info.num_cores, info.num_subcores
    nw = nc * ns
    chunk = 80
    n_chunks_per_worker = n_rows // (nw * chunk)
    assert n_chunks_per_worker * nw * chunk == n_rows

    sc = _sc_gather_add(n_rows, chunk, n_chunks_per_worker, nc, ns)
    out = sc(pcard, comb,
             card_ids.reshape(n_rows // chunk, chunk).astype(jnp.int32),
             cidx.reshape(n_rows // chunk, chunk))
    return out.reshape(B, L, EMB)
